# Initial kernel scaffold; baseline (speedup 1.0000x reference)
import jax
import jax.numpy as jnp
from jax.experimental import pallas as pl


def kernel(input, table):
    raise NotImplementedError("write your pallas kernel here")


# speedup vs baseline: 2.0354x; 2.0354x over previous
import functools

import jax
import jax.numpy as jnp
from jax import lax
from jax.experimental import pallas as pl
from jax.experimental.pallas import tpu as pltpu
from jax.experimental.pallas import tpu_sc as plsc

V = 100000
E = 300
EP = 304
B = 4096
L = 200
NCH = EP // 16
NC, NS = 2, 16
NW = NC * NS
SEQ_PER_W = B // NW


def _augment(table):
    vb = 2000

    def body(t_ref, o_ref):
        x = t_ref[...]
        flag = (jnp.max(jnp.abs(x), axis=1, keepdims=True) > 0.0)
        flag = flag.astype(jnp.float32)
        pad = jnp.zeros((vb, EP - E - 1), jnp.float32)
        o_ref[...] = jnp.concatenate([x, flag, pad], axis=1)

    return pl.pallas_call(
        body,
        grid=(V // vb,),
        in_specs=[pl.BlockSpec((vb, E), lambda i: (i, 0))],
        out_specs=pl.BlockSpec((vb, EP), lambda i: (i, 0)),
        out_shape=jax.ShapeDtypeStruct((V, EP), jnp.float32),
    )(table)


def _bag(aug, idx):
    mesh = plsc.VectorSubcoreMesh(core_axis_name="c", subcore_axis_name="s")

    @functools.partial(
        pl.kernel,
        out_type=jax.ShapeDtypeStruct((B, EP), jnp.float32),
        mesh=mesh,
        compiler_params=pltpu.CompilerParams(use_tc_tiling_on_sc=False),
        scratch_types=[
            pltpu.VMEM((L,), jnp.int32),
            pltpu.VMEM((L, EP), jnp.float32),
            pltpu.VMEM((EP,), jnp.float32),
            pltpu.SemaphoreType.DMA,
        ],
    )
    def k(aug_hbm, idx_hbm, out_hbm, idx_v, rows_v, res_v, sem):
        wid = lax.axis_index("s") * NC + lax.axis_index("c")
        base = wid * SEQ_PER_W

        @pl.loop(0, SEQ_PER_W)
        def _(i):
            b = base + i
            pltpu.sync_copy(idx_hbm.at[b], idx_v)
            pltpu.async_copy(aug_hbm.at[idx_v], rows_v, sem).wait()

            def body(j, accs):
                return tuple(
                    accs[c] + rows_v[j, pl.ds(c * 16, 16)] for c in range(NCH)
                )

            zero = jnp.zeros((16,), jnp.float32)
            accs = lax.fori_loop(0, L, body, tuple(zero for _ in range(NCH)))
            cnt = accs[NCH - 1][E % 16]
            inv = 1.0 / jnp.maximum(jnp.full((16,), cnt), 1.0)
            for c in range(NCH):
                res_v[pl.ds(c * 16, 16)] = accs[c] * inv
            pltpu.sync_copy(res_v, out_hbm.at[b])

    return k(aug, idx)


def kernel(input, table):
    idx = input.astype(jnp.int32)
    aug = _augment(table)
    out = _bag(aug, idx)
    return out[:, :E]


# speedup vs baseline: 2.8785x; 1.4142x over previous
import functools

import jax
import jax.numpy as jnp
from jax import lax
from jax.experimental import pallas as pl
from jax.experimental.pallas import tpu as pltpu
from jax.experimental.pallas import tpu_sc as plsc

V = 100000
E = 300
EP = 304
B = 4096
L = 200
NCH = EP // 16
NC, NS = 2, 16
NW = NC * NS
SEQ_PER_W = B // NW


def _augment(table):
    vb = 2000

    def body(t_ref, o_ref):
        x = t_ref[...]
        flag = (jnp.max(jnp.abs(x), axis=1, keepdims=True) > 0.0)
        flag = flag.astype(jnp.float32)
        pad = jnp.zeros((vb, EP - E - 1), jnp.float32)
        o_ref[...] = jnp.concatenate([x, flag, pad], axis=1)

    return pl.pallas_call(
        body,
        grid=(V // vb,),
        in_specs=[pl.BlockSpec((vb, E), lambda i: (i, 0))],
        out_specs=pl.BlockSpec((vb, EP), lambda i: (i, 0)),
        out_shape=jax.ShapeDtypeStruct((V, EP), jnp.float32),
    )(table)


def _bag(aug, idx):
    mesh = plsc.VectorSubcoreMesh(core_axis_name="c", subcore_axis_name="s")

    nbuf = 2

    @functools.partial(
        pl.kernel,
        out_type=jax.ShapeDtypeStruct((B, EP), jnp.float32),
        mesh=mesh,
        compiler_params=pltpu.CompilerParams(use_tc_tiling_on_sc=False),
        scratch_types=[
            pltpu.VMEM((nbuf, L), jnp.int32),
            pltpu.VMEM((nbuf, L, EP), jnp.float32),
            pltpu.VMEM((EP,), jnp.float32),
            pltpu.SemaphoreType.DMA,
            pltpu.SemaphoreType.DMA,
        ],
    )
    def k(aug_hbm, idx_hbm, out_hbm, idx_v, rows_v, res_v, sem0, sem1):
        wid = lax.axis_index("s") * NC + lax.axis_index("c")
        base = wid * SEQ_PER_W
        sems = (sem0, sem1)

        def start(kb, b):
            pltpu.sync_copy(idx_hbm.at[b], idx_v.at[kb])
            pltpu.async_copy(aug_hbm.at[idx_v.at[kb]], rows_v.at[kb], sems[kb])

        def finish(kb, b):
            pltpu.make_async_copy(
                aug_hbm.at[idx_v.at[kb]], rows_v.at[kb], sems[kb]
            ).wait()
            buf = rows_v.at[kb]

            def body(j, accs):
                return tuple(
                    accs[c] + buf[j, pl.ds(c * 16, 16)] for c in range(NCH)
                )

            zero = jnp.zeros((16,), jnp.float32)
            accs = lax.fori_loop(0, L, body, tuple(zero for _ in range(NCH)))
            cnt = accs[NCH - 1][E % 16]
            inv = 1.0 / jnp.maximum(jnp.full((16,), cnt), 1.0)
            for c in range(NCH):
                res_v[pl.ds(c * 16, 16)] = accs[c] * inv
            pltpu.sync_copy(res_v, out_hbm.at[b])

        for kb in range(nbuf):
            start(kb, base + kb)

        @pl.loop(0, SEQ_PER_W, step=nbuf)
        def _(i):
            for kb in range(nbuf):
                b = base + i + kb
                finish(kb, b)

                @pl.when(i + kb + nbuf < SEQ_PER_W)
                def _():
                    start(kb, b + nbuf)

    return k(aug, idx)


def kernel(input, table):
    idx = input.astype(jnp.int32)
    aug = _augment(table)
    out = _bag(aug, idx)
    return out[:, :E]
